# baseline (speedup 1.0000x reference)
import functools

import jax
import jax.numpy as jnp
import numpy as np
from jax import lax
from jax.experimental import pallas as pl
from jax.experimental.pallas import tpu as pltpu
from jax.experimental.pallas import tpu_sc as plsc

D_MODEL = 768
VOCAB = 100000
MAX_POS = 2048
BATCH = 4
SEQ = 2048

NC, NS, LANES = 2, 16, 16
NW = NC * NS
SPW = SEQ // NW
VPR = D_MODEL // LANES

CHUNK = 32
NCHUNK = BATCH * SPW // CHUNK
NBUF = 3
AHEAD = 2

SCALE = float(np.float32(np.sqrt(np.float32(D_MODEL))))


def _pos_encoding_np():
    pos = np.arange(MAX_POS)[:, np.newaxis]
    i = np.arange(D_MODEL)[np.newaxis, :]
    angle_rates = 1 / np.power(10000, 2 * i // np.float32(D_MODEL))
    angle_rads = pos * angle_rates
    angle_rads[:, 0::2] = np.sin(angle_rads[:, 0::2])
    angle_rads[:, 1::2] = np.cos(angle_rads[:, 1::2])
    return angle_rads.astype(np.float32)


_MESH = plsc.VectorSubcoreMesh(core_axis_name="c", subcore_axis_name="s")


@functools.partial(
    pl.kernel,
    out_type=jax.ShapeDtypeStruct((BATCH, SEQ, D_MODEL), jnp.float32),
    mesh=_MESH,
    scratch_types=[
        pltpu.VMEM((BATCH, SPW), jnp.int32),
        pltpu.VMEM((NBUF, CHUNK, D_MODEL), jnp.float32),
        pltpu.VMEM((SPW, D_MODEL), jnp.float32),
    ] + [pltpu.SemaphoreType.DMA] * (2 * NBUF),
)
def _emb_kernel(idx_hbm, table_hbm, pos_hbm, out_hbm,
                idx_v, rows_v, pos_v, *sems):
    gsem, ssem = sems[:NBUF], sems[NBUF:]
    wid = lax.axis_index("s") * NC + lax.axis_index("c")
    base = wid * SPW

    pltpu.sync_copy(pos_hbm.at[pl.ds(base, SPW), :], pos_v)
    for b in range(BATCH):
        pltpu.sync_copy(idx_hbm.at[b, pl.ds(base, SPW)], idx_v.at[b])

    def start_gather(i):
        b, off = divmod(i * CHUNK, SPW)
        return pltpu.async_copy(
            table_hbm.at[idx_v.at[b, pl.ds(off, CHUNK)]],
            rows_v.at[i % NBUF], gsem[i % NBUF])

    gathers = [None] * NCHUNK
    stores = [None] * NCHUNK
    for i in range(min(AHEAD, NCHUNK)):
        gathers[i] = start_gather(i)

    for i in range(NCHUNK):
        gathers[i].wait()
        b, off = divmod(i * CHUNK, SPW)
        buf = rows_v.at[i % NBUF]

        def row_body(r, _, buf=buf, off=off):
            for c in range(VPR):
                sl = pl.ds(c * LANES, LANES)
                buf[r, sl] = buf[r, sl] * SCALE + pos_v[r + off, sl]
            return 0

        lax.fori_loop(0, CHUNK, row_body, 0)
        stores[i] = pltpu.async_copy(
            buf, out_hbm.at[b, pl.ds(base + off, CHUNK), :], ssem[i % NBUF])

        nxt = i + AHEAD
        if nxt < NCHUNK:
            prev = nxt - NBUF
            if prev >= 0:
                stores[prev].wait()
                stores[prev] = None
            gathers[nxt] = start_gather(nxt)

    for s in stores:
        if s is not None:
            s.wait()


def kernel(inputs, table):
    pos = jnp.asarray(_pos_encoding_np())
    return _emb_kernel(inputs, table, pos)


# speedup vs baseline: 1.3083x; 1.3083x over previous
import functools

import jax
import jax.numpy as jnp
import numpy as np
from jax import lax
from jax.experimental import pallas as pl
from jax.experimental.pallas import tpu as pltpu
from jax.experimental.pallas import tpu_sc as plsc

D_MODEL = 768
VOCAB = 100000
MAX_POS = 2048
BATCH = 4
SEQ = 2048

NC, NS, LANES = 2, 16, 16
NW = NC * NS
SPW = SEQ // NW
VPR = D_MODEL // LANES

CHUNK = 32
NCHUNK = BATCH * SPW // CHUNK
NBUF = 3
AHEAD = 2

SCALE = float(np.float32(np.sqrt(np.float32(D_MODEL))))


def _pos_encoding_np():
    pos = np.arange(MAX_POS)[:, np.newaxis]
    i = np.arange(D_MODEL)[np.newaxis, :]
    angle_rates = 1 / np.power(10000, 2 * i // np.float32(D_MODEL))
    angle_rads = pos * angle_rates
    angle_rads[:, 0::2] = np.sin(angle_rads[:, 0::2])
    angle_rads[:, 1::2] = np.cos(angle_rads[:, 1::2])
    return angle_rads.astype(np.float32)


_MESH = plsc.VectorSubcoreMesh(core_axis_name="c", subcore_axis_name="s")


@functools.partial(
    pl.kernel,
    out_type=jax.ShapeDtypeStruct((BATCH, SEQ, D_MODEL), jnp.float32),
    mesh=_MESH,
    scratch_types=[
        pltpu.VMEM((BATCH, SPW), jnp.int32),
        pltpu.VMEM((NBUF, CHUNK, D_MODEL), jnp.float32),
        pltpu.VMEM((SPW, D_MODEL), jnp.float32),
    ] + [pltpu.SemaphoreType.DMA] * (2 * NBUF),
)
def _emb_kernel(idx_hbm, table_hbm, pos_hbm, out_hbm,
                idx_v, rows_v, pos_v, *sems):
    gsem, ssem = sems[:NBUF], sems[NBUF:]
    wid = lax.axis_index("s") * NC + lax.axis_index("c")
    base = wid * SPW

    pltpu.sync_copy(pos_hbm.at[pl.ds(base, SPW), :], pos_v)
    for b in range(BATCH):
        pltpu.sync_copy(idx_hbm.at[b, pl.ds(base, SPW)], idx_v.at[b])

    def start_gather(i):
        b, off = divmod(i * CHUNK, SPW)
        return pltpu.async_copy(
            table_hbm.at[idx_v.at[b, pl.ds(off, CHUNK)]],
            rows_v.at[i % NBUF], gsem[i % NBUF])

    gathers = [None] * NCHUNK
    stores = [None] * NCHUNK
    for i in range(min(AHEAD, NCHUNK)):
        gathers[i] = start_gather(i)

    for i in range(NCHUNK):
        gathers[i].wait()
        b, off = divmod(i * CHUNK, SPW)
        buf = rows_v.at[i % NBUF]

        @plsc.parallel_loop(0, CHUNK)
        def row_body(r, buf=buf, off=off):
            for c in range(VPR):
                sl = pl.ds(c * LANES, LANES)
                buf[r, sl] = buf[r, sl] * SCALE + pos_v[r + off, sl]
        stores[i] = pltpu.async_copy(
            buf, out_hbm.at[b, pl.ds(base + off, CHUNK), :], ssem[i % NBUF])

        nxt = i + AHEAD
        if nxt < NCHUNK:
            prev = nxt - NBUF
            if prev >= 0:
                stores[prev].wait()
                stores[prev] = None
            gathers[nxt] = start_gather(nxt)

    for s in stores:
        if s is not None:
            s.wait()


def kernel(inputs, table):
    pos = jnp.asarray(_pos_encoding_np())
    return _emb_kernel(inputs, table, pos)
